# Initial kernel scaffold; baseline (speedup 1.0000x reference)
import jax
import jax.numpy as jnp
from jax.experimental import pallas as pl


def kernel(token_ids, embed_matrix):
    raise NotImplementedError("write your pallas kernel here")


# speedup vs baseline: 1.0940x; 1.0940x over previous
import functools

import jax
import jax.numpy as jnp
from jax import lax
from jax.experimental import pallas as pl
from jax.experimental.pallas import tpu as pltpu
from jax.experimental.pallas import tpu_sc as plsc

_NUM_CORES = 2
_NUM_SUBCORES = 16
_NUM_WORKERS = _NUM_CORES * _NUM_SUBCORES
_CHUNK = 1024


@functools.partial(jax.jit, static_argnums=(2, 3))
def _sc_gather(flat_ids, table, b_padded, d):
    b_per_w = b_padded // _NUM_WORKERS
    n_chunks = b_per_w // _CHUNK
    mesh = plsc.VectorSubcoreMesh(core_axis_name="c", subcore_axis_name="s")

    @functools.partial(
        pl.kernel,
        mesh=mesh,
        out_type=jax.ShapeDtypeStruct((b_padded, d), jnp.float32),
        scratch_types=[
            pltpu.VMEM((_CHUNK,), jnp.int32),
            pltpu.VMEM((_CHUNK, d), jnp.float32),
            pltpu.SemaphoreType.DMA,
        ],
        compiler_params=pltpu.CompilerParams(use_tc_tiling_on_sc=False),
    )
    def k(idx_hbm, table_hbm, out_hbm, idx_v, rows_v, sem):
        wid = lax.axis_index("s") * _NUM_CORES + lax.axis_index("c")
        base = wid * b_per_w

        def body(g, carry):
            off = base + g * _CHUNK
            pltpu.sync_copy(idx_hbm.at[pl.ds(off, _CHUNK)], idx_v)
            pltpu.async_copy(table_hbm.at[idx_v], rows_v, sem).wait()
            pltpu.sync_copy(rows_v, out_hbm.at[pl.ds(off, _CHUNK)])
            return carry

        lax.fori_loop(0, n_chunks, body, 0)

    return k(flat_ids, table)


def kernel(token_ids, embed_matrix):
    orig_shape = token_ids.shape
    d = embed_matrix.shape[1]
    flat = token_ids.reshape(-1).astype(jnp.int32)
    b = flat.shape[0]
    granularity = _NUM_WORKERS * _CHUNK
    b_padded = ((b + granularity - 1) // granularity) * granularity
    if b_padded != b:
        flat = jnp.pad(flat, (0, b_padded - b))
    out = _sc_gather(flat, embed_matrix, b_padded, d)
    return out[:b].reshape(*orig_shape, d)


# speedup vs baseline: 1.1122x; 1.0166x over previous
import functools

import jax
import jax.numpy as jnp
from jax import lax
from jax.experimental import pallas as pl
from jax.experimental.pallas import tpu as pltpu
from jax.experimental.pallas import tpu_sc as plsc

_NUM_CORES = 2
_NUM_SUBCORES = 16
_NUM_WORKERS = _NUM_CORES * _NUM_SUBCORES
_CHUNK = 640
_NBUF = 4
_LAG = 2


@functools.partial(jax.jit, static_argnums=(2, 3))
def _sc_gather(flat_ids, table, b_padded, d):
    b_per_w = b_padded // _NUM_WORKERS
    n_chunks = b_per_w // _CHUNK
    mesh = plsc.VectorSubcoreMesh(core_axis_name="c", subcore_axis_name="s")

    @functools.partial(
        pl.kernel,
        mesh=mesh,
        out_type=jax.ShapeDtypeStruct((b_padded, d), jnp.float32),
        scratch_types=[pltpu.VMEM((_CHUNK,), jnp.int32)] * _NBUF
        + [pltpu.VMEM((_CHUNK, d), jnp.float32)] * _NBUF
        + [pltpu.SemaphoreType.DMA] * (3 * _NBUF),
        compiler_params=pltpu.CompilerParams(use_tc_tiling_on_sc=False),
    )
    def k(idx_hbm, table_hbm, out_hbm, *bufs):
        idx_v = bufs[:_NBUF]
        rows_v = bufs[_NBUF:2 * _NBUF]
        sems = bufs[2 * _NBUF:]
        sem_i = sems[:_NBUF]
        sem_g = sems[_NBUF:2 * _NBUF]
        sem_s = sems[2 * _NBUF:]
        wid = lax.axis_index("s") * _NUM_CORES + lax.axis_index("c")
        base = wid * b_per_w

        def idx_copy(g):
            s = g % _NBUF
            return pltpu.make_async_copy(
                idx_hbm.at[pl.ds(base + g * _CHUNK, _CHUNK)],
                idx_v[s], sem_i[s])

        def gather_copy(g):
            s = g % _NBUF
            return pltpu.make_async_copy(
                table_hbm.at[idx_v[s]], rows_v[s], sem_g[s])

        def store_copy(g):
            s = g % _NBUF
            return pltpu.make_async_copy(
                rows_v[s],
                out_hbm.at[pl.ds(base + g * _CHUNK, _CHUNK)], sem_s[s])

        for g in range(min(_NBUF, n_chunks)):
            idx_copy(g).start()
        for g in range(n_chunks):
            idx_copy(g).wait()
            if g >= _NBUF:
                store_copy(g - _NBUF).wait()
            gather_copy(g).start()
            h = g - _LAG
            if h >= 0:
                gather_copy(h).wait()
                store_copy(h).start()
                if h + _NBUF < n_chunks:
                    idx_copy(h + _NBUF).start()
        for h in range(max(0, n_chunks - _LAG), n_chunks):
            gather_copy(h).wait()
            store_copy(h).start()
        for g in range(max(0, n_chunks - _NBUF), n_chunks):
            store_copy(g).wait()

    return k(flat_ids, table)


def kernel(token_ids, embed_matrix):
    orig_shape = token_ids.shape
    d = embed_matrix.shape[1]
    flat = token_ids.reshape(-1).astype(jnp.int32)
    b = flat.shape[0]
    granularity = _NUM_WORKERS * _CHUNK
    b_padded = ((b + granularity - 1) // granularity) * granularity
    if b_padded != b:
        flat = jnp.pad(flat, (0, b_padded - b))
    out = _sc_gather(flat, embed_matrix, b_padded, d)
    return out[:b].reshape(*orig_shape, d)


# speedup vs baseline: 1.5519x; 1.3954x over previous
import functools

import jax
import jax.numpy as jnp
from jax import lax
from jax.experimental import pallas as pl
from jax.experimental.pallas import tpu as pltpu
from jax.experimental.pallas import tpu_sc as plsc

_NUM_CORES = 2
_NUM_SUBCORES = 16
_NUM_WORKERS = _NUM_CORES * _NUM_SUBCORES
_CHUNK = 256
_LANES = 16


@functools.partial(jax.jit, static_argnums=(3, 4, 5))
def _sc_gather(sup_ids, col_offs, table_sr, n_s, n_b, d):
    b_per_w = n_b // _NUM_WORKERS
    chunks_per_row = b_per_w // _CHUNK
    n_chunks = n_s * chunks_per_row
    mesh = plsc.VectorSubcoreMesh(core_axis_name="c", subcore_axis_name="s")

    @functools.partial(
        pl.kernel,
        mesh=mesh,
        out_type=jax.ShapeDtypeStruct((n_s, d, n_b), jnp.float32),
        scratch_types=[pltpu.VMEM((_CHUNK,), jnp.int32)] * 2
        + [pltpu.VMEM((_CHUNK,), jnp.int32)] * 2
        + [pltpu.VMEM((_CHUNK, 128), jnp.float32)] * 2
        + [pltpu.VMEM((d, _CHUNK), jnp.float32)] * 2
        + [pltpu.SemaphoreType.DMA] * 8,
        compiler_params=pltpu.CompilerParams(needs_layout_passes=False),
    )
    def k(sup_hbm, col_hbm, table_hbm, out_hbm, *refs):
        sup_v = refs[0:2]
        col_v = refs[2:4]
        rows_v = refs[4:6]
        slab_v = refs[6:8]
        sem_i = refs[8:10]
        sem_g = refs[10:12]
        sem_s = refs[12:14]
        sem_x = refs[14:16]
        wid = lax.axis_index("s") * _NUM_CORES + lax.axis_index("c")
        bw0 = wid * b_per_w

        def flat_off(c):
            s = c // chunks_per_row
            h = c % chunks_per_row
            boff = bw0 + h * _CHUNK
            return s, boff, s * n_b + boff

        def ids_start(c, slot):
            _, _, p0 = flat_off(c)
            pltpu.make_async_copy(
                sup_hbm.at[pl.ds(p0, _CHUNK)], sup_v[slot], sem_i[slot]
            ).start()
            pltpu.make_async_copy(
                col_hbm.at[pl.ds(p0, _CHUNK)], col_v[slot], sem_x[slot]
            ).start()

        def ids_wait(slot):
            pltpu.make_async_copy(
                sup_hbm.at[pl.ds(0, _CHUNK)], sup_v[slot], sem_i[slot]
            ).wait()
            pltpu.make_async_copy(
                col_hbm.at[pl.ds(0, _CHUNK)], col_v[slot], sem_x[slot]
            ).wait()

        def gather(slot):
            return pltpu.make_async_copy(
                table_hbm.at[sup_v[slot]], rows_v[slot], sem_g[slot])

        def store(c, slot):
            s, boff, _ = flat_off(c)
            if not isinstance(boff, int):
                boff = pl.multiple_of(boff, _CHUNK)
            return pltpu.make_async_copy(
                slab_v[slot],
                out_hbm.at[s, :, pl.ds(boff, _CHUNK)], sem_s[slot])

        def extract(slot):
            rows = rows_v[slot]
            col = col_v[slot]
            slab = slab_v[slot]

            def jb_body(jb, carry):
                j0 = jb * _LANES
                jv = lax.iota(jnp.int32, _LANES) + j0
                cv = col[pl.ds(j0, _LANES)]
                for dd in range(d):
                    slab[dd, pl.ds(j0, _LANES)] = plsc.load_gather(
                        rows, [jv, cv + dd])
                return carry

            lax.fori_loop(0, _CHUNK // _LANES, jb_body, 0)

        ids_start(0, 0)
        ids_start(1, 1)
        ids_wait(0)
        gather(0).start()

        def pair_body(i, carry):
            a = 2 * i
            gather(0).wait()
            ids_wait(1)
            gather(1).start()

            @pl.when(i > 0)
            def _():
                store(0, 0).wait()

            extract(0)
            store(a, 0).start()

            @pl.when(a + 2 < n_chunks)
            def _():
                ids_start(a + 2, 0)

            gather(1).wait()

            @pl.when(a + 2 < n_chunks)
            def _():
                ids_wait(0)
                gather(0).start()

            @pl.when(i > 0)
            def _():
                store(0, 1).wait()

            extract(1)
            store(a + 1, 1).start()

            @pl.when(a + 3 < n_chunks)
            def _():
                ids_start(a + 3, 1)

            return carry

        lax.fori_loop(0, n_chunks // 2, pair_body, 0)
        store(n_chunks - 2, 0).wait()
        store(n_chunks - 1, 1).wait()

    return k(sup_ids, col_offs, table_sr)


def kernel(token_ids, embed_matrix):
    n_rows, n_cols = token_ids.shape
    v, d = embed_matrix.shape
    flat = jnp.swapaxes(token_ids, 0, 1).reshape(-1).astype(jnp.int32)
    per_sr = 128 // d
    sup = flat // per_sr
    col = (flat % per_sr) * d
    table_sr = embed_matrix.reshape(v // per_sr, 128)
    out = _sc_gather(sup, col, table_sr, n_cols, n_rows, d)
    return jnp.transpose(out, (2, 0, 1))


# speedup vs baseline: 1.6305x; 1.0506x over previous
import functools

import jax
import jax.numpy as jnp
from jax import lax
from jax.experimental import pallas as pl
from jax.experimental.pallas import tpu as pltpu
from jax.experimental.pallas import tpu_sc as plsc

_NUM_CORES = 2
_NUM_SUBCORES = 16
_NUM_WORKERS = _NUM_CORES * _NUM_SUBCORES
_CHUNK = 128
_NBUF = 4
_LANES = 16


@functools.partial(jax.jit, static_argnums=(2, 3, 4))
def _sc_gather(flat_ids, table_sr, n_s, n_b, d):
    per_sr = 128 // d
    sr_shift = per_sr.bit_length() - 1
    b_per_w = n_b // _NUM_WORKERS
    chunks_per_row = b_per_w // _CHUNK
    n_chunks = n_s * chunks_per_row
    mesh = plsc.VectorSubcoreMesh(core_axis_name="c", subcore_axis_name="s")

    @functools.partial(
        pl.kernel,
        mesh=mesh,
        out_type=jax.ShapeDtypeStruct((n_s, d, n_b), jnp.float32),
        scratch_types=[pltpu.VMEM((_CHUNK,), jnp.int32)] * _NBUF
        + [pltpu.VMEM((_CHUNK,), jnp.int32)] * _NBUF
        + [pltpu.VMEM((_CHUNK, 128), jnp.float32)] * _NBUF
        + [pltpu.VMEM((d, _CHUNK), jnp.float32)] * _NBUF
        + [pltpu.SemaphoreType.DMA] * (3 * _NBUF),
        compiler_params=pltpu.CompilerParams(needs_layout_passes=False),
    )
    def k(ids_hbm, table_hbm, out_hbm, *refs):
        ids_v = refs[0:_NBUF]
        sup_v = refs[_NBUF:2 * _NBUF]
        rows_v = refs[2 * _NBUF:3 * _NBUF]
        slab_v = refs[3 * _NBUF:4 * _NBUF]
        sems = refs[4 * _NBUF:]
        sem_i = sems[0:_NBUF]
        sem_g = sems[_NBUF:2 * _NBUF]
        sem_s = sems[2 * _NBUF:]
        wid = lax.axis_index("s") * _NUM_CORES + lax.axis_index("c")
        bw0 = wid * b_per_w

        def flat_off(c):
            s = c // chunks_per_row
            h = c % chunks_per_row
            boff = bw0 + h * _CHUNK
            return s, boff, s * n_b + boff

        def ids_copy(c, slot):
            _, _, p0 = flat_off(c)
            return pltpu.make_async_copy(
                ids_hbm.at[pl.ds(p0, _CHUNK)], ids_v[slot], sem_i[slot])

        def sup_compute(slot):
            ids = ids_v[slot]
            sup = sup_v[slot]
            for jb in range(_CHUNK // _LANES):
                j0 = jb * _LANES
                sup[pl.ds(j0, _LANES)] = lax.shift_right_logical(
                    ids[pl.ds(j0, _LANES)], sr_shift)

        def gather(slot):
            return pltpu.make_async_copy(
                table_hbm.at[sup_v[slot]], rows_v[slot], sem_g[slot])

        def store(c, slot):
            s, boff, _ = flat_off(c)
            if not isinstance(boff, int):
                boff = pl.multiple_of(boff, _CHUNK)
            return pltpu.make_async_copy(
                slab_v[slot],
                out_hbm.at[s, :, pl.ds(boff, _CHUNK)], sem_s[slot])

        def extract(slot):
            rows = rows_v[slot]
            ids = ids_v[slot]
            slab = slab_v[slot]

            def jb_body(jb, carry):
                j0 = jb * _LANES
                jv = lax.iota(jnp.int32, _LANES) + j0
                cv = (ids[pl.ds(j0, _LANES)] & (per_sr - 1)) * d
                for dd in range(d):
                    slab[dd, pl.ds(j0, _LANES)] = plsc.load_gather(
                        rows, [jv, cv + dd])
                return carry

            lax.fori_loop(0, _CHUNK // _LANES, jb_body, 0)

        for c in range(_NBUF):
            ids_copy(c, c).start()
        for c in range(2):
            ids_copy(c, c).wait()
            sup_compute(c)
            gather(c).start()

        def chunk_step(g, slot):
            gather(slot).wait()

            @pl.when(g + 2 < n_chunks)
            def _():
                nxt = (slot + 2) % _NBUF
                ids_copy(0, nxt).wait()
                sup_compute(nxt)
                gather(nxt).start()

            @pl.when(g >= _NBUF)
            def _():
                store(0, slot).wait()

            extract(slot)
            store(g, slot).start()
            @pl.when(g + _NBUF < n_chunks)
            def _():
                ids_copy(g + _NBUF, slot).start()

        def quad_body(i, carry):
            for k in range(_NBUF):
                chunk_step(i * _NBUF + k, k)
            return carry

        lax.fori_loop(0, n_chunks // _NBUF, quad_body, 0)
        for c in range(n_chunks - _NBUF, n_chunks):
            store(0, c % _NBUF).wait()

    return k(flat_ids, table_sr)


def kernel(token_ids, embed_matrix):
    n_rows, n_cols = token_ids.shape
    v, d = embed_matrix.shape
    per_sr = 128 // d
    flat = jnp.swapaxes(token_ids, 0, 1).reshape(-1).astype(jnp.int32)
    table_sr = embed_matrix.reshape(v // per_sr, 128)
    out = _sc_gather(flat, table_sr, n_cols, n_rows, d)
    return jnp.transpose(out, (2, 0, 1))


# speedup vs baseline: 2.5664x; 1.5740x over previous
import functools

import jax
import jax.numpy as jnp
from jax import lax
from jax.experimental import pallas as pl
from jax.experimental.pallas import tpu as pltpu
from jax.experimental.pallas import tpu_sc as plsc

_NUM_CORES = 2
_NUM_SUBCORES = 16
_NUM_WORKERS = _NUM_CORES * _NUM_SUBCORES
_CHUNK = 128
_NBUF = 4
_LANES = 16


@functools.partial(jax.jit, static_argnums=(2, 3, 4))
def _sc_gather(flat_ids, table_sr, n_s, n_b, d):
    per_sr = 128 // d
    sr_shift = per_sr.bit_length() - 1
    b_per_w = n_b // _NUM_WORKERS
    chunks_per_row = b_per_w // _CHUNK
    n_chunks = n_s * chunks_per_row
    mesh = plsc.VectorSubcoreMesh(core_axis_name="c", subcore_axis_name="s")

    @functools.partial(
        pl.kernel,
        mesh=mesh,
        out_type=jax.ShapeDtypeStruct((n_s, d, n_b), jnp.float32),
        scratch_types=[pltpu.VMEM((_CHUNK,), jnp.int32)] * _NBUF
        + [pltpu.VMEM((_CHUNK,), jnp.int32)] * _NBUF
        + [pltpu.VMEM((_CHUNK, 128), jnp.float32)] * _NBUF
        + [pltpu.VMEM((d, _CHUNK), jnp.float32)] * _NBUF
        + [pltpu.SemaphoreType.DMA] * (3 * _NBUF),
        compiler_params=pltpu.CompilerParams(needs_layout_passes=False),
    )
    def k(ids_hbm, table_hbm, out_hbm, *refs):
        ids_v = refs[0:_NBUF]
        sup_v = refs[_NBUF:2 * _NBUF]
        rows_v = refs[2 * _NBUF:3 * _NBUF]
        slab_v = refs[3 * _NBUF:4 * _NBUF]
        sems = refs[4 * _NBUF:]
        sem_i = sems[0:_NBUF]
        sem_g = sems[_NBUF:2 * _NBUF]
        sem_s = sems[2 * _NBUF:]
        wid = lax.axis_index("s") * _NUM_CORES + lax.axis_index("c")
        bw0 = wid * b_per_w

        def flat_off(c):
            s = c // chunks_per_row
            h = c % chunks_per_row
            boff = bw0 + h * _CHUNK
            return s, boff, s * n_b + boff

        def ids_copy(c, slot):
            _, _, p0 = flat_off(c)
            return pltpu.make_async_copy(
                ids_hbm.at[pl.ds(p0, _CHUNK)], ids_v[slot], sem_i[slot])

        def sup_compute(slot):
            ids = ids_v[slot]
            sup = sup_v[slot]
            for jb in range(_CHUNK // _LANES):
                j0 = jb * _LANES
                sup[pl.ds(j0, _LANES)] = lax.shift_right_logical(
                    ids[pl.ds(j0, _LANES)], sr_shift)

        def gather(slot):
            return pltpu.make_async_copy(
                table_hbm.at[sup_v[slot]], rows_v[slot], sem_g[slot])

        def store(c, slot):
            s, boff, _ = flat_off(c)
            if not isinstance(boff, int):
                boff = pl.multiple_of(boff, _CHUNK)
            return pltpu.make_async_copy(
                slab_v[slot],
                out_hbm.at[s, :, pl.ds(boff, _CHUNK)], sem_s[slot])

        def extract(slot):
            rows = rows_v[slot]
            ids = ids_v[slot]
            slab = slab_v[slot]
            lanes = lax.iota(jnp.int32, _LANES)

            def jb_body(jb, carry):
                j0 = jb * _LANES
                jv = lanes + j0
                cv = (ids[pl.ds(j0, _LANES)] & (per_sr - 1)) * d
                for a in range(d):
                    dv = (lanes + a) & (d - 1)
                    vals = plsc.load_gather(rows, [jv, cv + dv])
                    plsc.store_scatter(slab, [dv, jv], vals)
                return carry

            lax.fori_loop(0, _CHUNK // _LANES, jb_body, 0)

        for c in range(_NBUF):
            ids_copy(c, c).start()
        for c in range(2):
            ids_copy(c, c).wait()
            sup_compute(c)
            gather(c).start()

        def chunk_step(g, slot):
            gather(slot).wait()

            @pl.when(g + 2 < n_chunks)
            def _():
                nxt = (slot + 2) % _NBUF
                ids_copy(0, nxt).wait()
                sup_compute(nxt)
                gather(nxt).start()

            @pl.when(g >= _NBUF)
            def _():
                store(0, slot).wait()

            extract(slot)
            store(g, slot).start()
            @pl.when(g + _NBUF < n_chunks)
            def _():
                ids_copy(g + _NBUF, slot).start()

        def quad_body(i, carry):
            for k in range(_NBUF):
                chunk_step(i * _NBUF + k, k)
            return carry

        lax.fori_loop(0, n_chunks // _NBUF, quad_body, 0)
        for c in range(n_chunks - _NBUF, n_chunks):
            store(0, c % _NBUF).wait()

    return k(flat_ids, table_sr)


def kernel(token_ids, embed_matrix):
    n_rows, n_cols = token_ids.shape
    v, d = embed_matrix.shape
    per_sr = 128 // d
    flat = jnp.swapaxes(token_ids, 0, 1).reshape(-1).astype(jnp.int32)
    table_sr = embed_matrix.reshape(v // per_sr, 128)
    out = _sc_gather(flat, table_sr, n_cols, n_rows, d)
    return jnp.transpose(out, (2, 0, 1))


# speedup vs baseline: 3.8396x; 1.4961x over previous
import functools

import jax
import jax.numpy as jnp
from jax import lax
from jax.experimental import pallas as pl
from jax.experimental.pallas import tpu as pltpu
from jax.experimental.pallas import tpu_sc as plsc

_NUM_CORES = 2
_NUM_SUBCORES = 16
_NUM_WORKERS = _NUM_CORES * _NUM_SUBCORES
_CHUNK = 128
_NBUF = 4
_LANES = 16


@functools.partial(jax.jit, static_argnums=(1,))
def _sc_transpose(table_t, d):
    v = table_t.shape[1]
    per_sr = 128 // d
    cw = 512
    n_full = v // cw
    tail = v - n_full * cw
    sr_per_chunk = cw // per_sr
    mesh = plsc.VectorSubcoreMesh(core_axis_name="c", subcore_axis_name="s")

    @functools.partial(
        pl.kernel,
        mesh=mesh,
        out_type=jax.ShapeDtypeStruct((v * d // 128, 128), jnp.float32),
        scratch_types=[pltpu.VMEM((d, cw), jnp.float32)] * 2
        + [pltpu.VMEM((sr_per_chunk, 128), jnp.float32)] * 2
        + [pltpu.VMEM((d, max(tail, per_sr)), jnp.float32),
           pltpu.VMEM((max(tail // per_sr, 1), 128), jnp.float32)]
        + [pltpu.SemaphoreType.DMA] * 6,
        compiler_params=pltpu.CompilerParams(needs_layout_passes=False),
    )
    def kt(in_hbm, out_hbm, *refs):
        bin_ = refs[0:2]
        bout = refs[2:4]
        tin, tout = refs[4], refs[5]
        sem_i = refs[6:8]
        sem_o = refs[8:10]
        sem_t = refs[10:12]
        wid = lax.axis_index("s") * _NUM_CORES + lax.axis_index("c")
        lanes = lax.iota(jnp.int32, _LANES)
        dims = [(lanes + a) & (d - 1) for a in range(d)]

        def in_copy(c, slot):
            off = pl.multiple_of(c * cw, cw)
            return pltpu.make_async_copy(
                in_hbm.at[:, pl.ds(off, cw)], bin_[slot], sem_i[slot])

        def out_copy(c, slot):
            off = pl.multiple_of(c * sr_per_chunk, sr_per_chunk)
            return pltpu.make_async_copy(
                bout[slot], out_hbm.at[pl.ds(off, sr_per_chunk)],
                sem_o[slot])

        def transpose(src, dst, n_cols):
            def cg_body(cg, carry):
                colv = lanes + cg * _LANES
                srv = lax.shift_right_logical(colv, sr_shift_l)
                cb = (colv & (per_sr - 1)) * d
                for a in range(d):
                    vals = plsc.load_gather(src, [dims[a], colv])
                    plsc.store_scatter(dst, [srv, cb + dims[a]], vals)
                return carry

            lax.fori_loop(0, n_cols // _LANES, cg_body, 0)

        sr_shift_l = per_sr.bit_length() - 1

        n_iter = (n_full + _NUM_WORKERS - 1) // _NUM_WORKERS
        n_pairs = (n_iter + 1) // 2

        def chunk_at(i, k):
            return (2 * i + k) * _NUM_WORKERS + wid

        @pl.when(chunk_at(0, 0) < n_full)
        def _():
            in_copy(chunk_at(0, 0), 0).start()

        @pl.when(chunk_at(0, 1) < n_full)
        def _():
            in_copy(chunk_at(0, 1), 1).start()

        def pair_body(i, carry):
            for k in range(2):
                c = chunk_at(i, k)

                @pl.when(c < n_full)
                def _():
                    in_copy(c, k).wait()

                    @pl.when(i > 0)
                    def _():
                        out_copy(0, k).wait()

                    transpose(bin_[k], bout[k], cw)
                    out_copy(c, k).start()
                    nc = chunk_at(i + 1, k)

                    @pl.when(nc < n_full)
                    def _():
                        in_copy(nc, k).start()

            return carry

        lax.fori_loop(0, n_pairs, pair_body, 0)
        for k in range(2):
            @pl.when(chunk_at(0, k) < n_full)
            def _():
                out_copy(0, k).wait()

        if tail:
            @pl.when(wid == _NUM_WORKERS - 1)
            def _():
                tc_in = pltpu.make_async_copy(
                    in_hbm.at[:, pl.ds(n_full * cw, tail)], tin, sem_t[0])
                tc_in.start()
                tc_in.wait()
                transpose(tin, tout, tail)
                tc_out = pltpu.make_async_copy(
                    tout,
                    out_hbm.at[pl.ds(n_full * sr_per_chunk,
                                     tail // per_sr)], sem_t[1])
                tc_out.start()
                tc_out.wait()

    return kt(table_t)


@functools.partial(jax.jit, static_argnums=(2, 3, 4))
def _sc_gather(flat_ids, table_sr, n_s, n_b, d):
    per_sr = 128 // d
    sr_shift = per_sr.bit_length() - 1
    b_per_w = n_b // _NUM_WORKERS
    chunks_per_row = b_per_w // _CHUNK
    n_chunks = n_s * chunks_per_row
    mesh = plsc.VectorSubcoreMesh(core_axis_name="c", subcore_axis_name="s")

    @functools.partial(
        pl.kernel,
        mesh=mesh,
        out_type=jax.ShapeDtypeStruct((n_s, d, n_b), jnp.float32),
        scratch_types=[pltpu.VMEM((_CHUNK,), jnp.int32)] * _NBUF
        + [pltpu.VMEM((_CHUNK,), jnp.int32)] * _NBUF
        + [pltpu.VMEM((_CHUNK, 128), jnp.float32)] * _NBUF
        + [pltpu.VMEM((d, _CHUNK), jnp.float32)] * _NBUF
        + [pltpu.SemaphoreType.DMA] * (3 * _NBUF),
        compiler_params=pltpu.CompilerParams(needs_layout_passes=False),
    )
    def k(ids_hbm, table_hbm, out_hbm, *refs):
        ids_v = refs[0:_NBUF]
        sup_v = refs[_NBUF:2 * _NBUF]
        rows_v = refs[2 * _NBUF:3 * _NBUF]
        slab_v = refs[3 * _NBUF:4 * _NBUF]
        sems = refs[4 * _NBUF:]
        sem_i = sems[0:_NBUF]
        sem_g = sems[_NBUF:2 * _NBUF]
        sem_s = sems[2 * _NBUF:]
        wid = lax.axis_index("s") * _NUM_CORES + lax.axis_index("c")
        bw0 = wid * b_per_w

        def flat_off(c):
            s = c // chunks_per_row
            h = c % chunks_per_row
            boff = bw0 + h * _CHUNK
            return s, boff, s * n_b + boff

        def ids_copy(c, slot):
            _, _, p0 = flat_off(c)
            return pltpu.make_async_copy(
                ids_hbm.at[pl.ds(p0, _CHUNK)], ids_v[slot], sem_i[slot])

        def sup_compute(slot):
            ids = ids_v[slot]
            sup = sup_v[slot]
            for jb in range(_CHUNK // _LANES):
                j0 = jb * _LANES
                sup[pl.ds(j0, _LANES)] = lax.shift_right_logical(
                    ids[pl.ds(j0, _LANES)], sr_shift)

        def gather(slot):
            return pltpu.make_async_copy(
                table_hbm.at[sup_v[slot]], rows_v[slot], sem_g[slot])

        def store(c, slot):
            s, boff, _ = flat_off(c)
            if not isinstance(boff, int):
                boff = pl.multiple_of(boff, _CHUNK)
            return pltpu.make_async_copy(
                slab_v[slot],
                out_hbm.at[s, :, pl.ds(boff, _CHUNK)], sem_s[slot])

        def extract(slot):
            rows = rows_v[slot]
            ids = ids_v[slot]
            slab = slab_v[slot]
            lanes = lax.iota(jnp.int32, _LANES)

            def jb_body(jb, carry):
                j0 = jb * _LANES
                jv = lanes + j0
                cv = (ids[pl.ds(j0, _LANES)] & (per_sr - 1)) * d
                for a in range(d):
                    dv = (lanes + a) & (d - 1)
                    vals = plsc.load_gather(rows, [jv, cv + dv])
                    plsc.store_scatter(slab, [dv, jv], vals)
                return carry

            lax.fori_loop(0, _CHUNK // _LANES, jb_body, 0)

        for c in range(_NBUF):
            ids_copy(c, c).start()
        for c in range(2):
            ids_copy(c, c).wait()
            sup_compute(c)
            gather(c).start()

        def chunk_step(g, slot):
            gather(slot).wait()

            @pl.when(g + 2 < n_chunks)
            def _():
                nxt = (slot + 2) % _NBUF
                ids_copy(0, nxt).wait()
                sup_compute(nxt)
                gather(nxt).start()

            @pl.when(g >= _NBUF)
            def _():
                store(0, slot).wait()

            extract(slot)
            store(g, slot).start()
            @pl.when(g + _NBUF < n_chunks)
            def _():
                ids_copy(g + _NBUF, slot).start()

        def quad_body(i, carry):
            for k in range(_NBUF):
                chunk_step(i * _NBUF + k, k)
            return carry

        lax.fori_loop(0, n_chunks // _NBUF, quad_body, 0)
        for c in range(n_chunks - _NBUF, n_chunks):
            store(0, c % _NBUF).wait()

    return k(flat_ids, table_sr)


def kernel(token_ids, embed_matrix):
    n_rows, n_cols = token_ids.shape
    v, d = embed_matrix.shape
    flat = jnp.swapaxes(token_ids, 0, 1).reshape(-1).astype(jnp.int32)
    table_sr = _sc_transpose(jnp.swapaxes(embed_matrix, 0, 1), d)
    out = _sc_gather(flat, table_sr, n_cols, n_rows, d)
    return jnp.transpose(out, (2, 0, 1))
